# baseline (speedup 1.0000x reference)
import functools

import jax
import jax.numpy as jnp
from jax import lax
from jax.experimental import pallas as pl
from jax.experimental.pallas import tpu as pltpu
from jax.experimental.pallas import tpu_sc as plsc

B, L = 16384, 200
NV = 10
D = 8
NCODE = NV * NV
LUT_W = 128


def _lutc_body(t1_ref, t2_ref, w_ref, b_ref, out_ref):
    w = w_ref[...]
    t1 = t1_ref[...]
    t2 = t2_ref[...]
    t1w = sum(t1[:, c:c + 1] * w[c:c + 1, :] for c in range(4))
    t2w = sum(t2[:, c:c + 1] * w[4 + c:5 + c, :] for c in range(4))
    p_row = lax.broadcasted_iota(jnp.int32, (NCODE, NV), 0)
    p_col = lax.broadcasted_iota(jnp.int32, (NCODE, NV), 1)
    e_div = jnp.where(p_row // NV == p_col, 1.0, 0.0)
    e_mod = jnp.where(p_row % NV == p_col, 1.0, 0.0)
    lutr = jnp.dot(e_div, t1w, preferred_element_type=jnp.float32)
    lutr = lutr + jnp.dot(e_mod, t2w, preferred_element_type=jnp.float32)
    lutr = jnp.maximum(lutr + b_ref[...], 0.0)
    eye = jnp.where(
        lax.broadcasted_iota(jnp.int32, (NCODE, NCODE), 0)
        == lax.broadcasted_iota(jnp.int32, (NCODE, NCODE), 1), 1.0, 0.0)
    lutc = lax.dot_general(lutr, eye, (((0,), (0,)), ((), ())),
                           preferred_element_type=jnp.float32)
    out_ref[...] = jnp.concatenate(
        [lutc, jnp.zeros((D, LUT_W - NCODE), jnp.float32)], axis=1)


_lutc_call = pl.pallas_call(
    _lutc_body,
    out_shape=jax.ShapeDtypeStruct((D, LUT_W), jnp.float32),
)

_IDX_BLK = 2048


def _idxT_body(i1_ref, i2_ref, out_ref):
    idxc = (i1_ref[...] * NV + i2_ref[...]).astype(jnp.bfloat16)
    eye = jnp.where(
        lax.broadcasted_iota(jnp.int32, (L, L), 0)
        == lax.broadcasted_iota(jnp.int32, (L, L), 1),
        1.0, 0.0).astype(jnp.bfloat16)
    out = lax.dot_general(eye, idxc, (((0,), (1,)), ((), ())),
                          preferred_element_type=jnp.float32)
    out_ref[...] = out.astype(jnp.int32)


_idxT_call = pl.pallas_call(
    _idxT_body,
    grid=(B // _IDX_BLK,),
    in_specs=[
        pl.BlockSpec((_IDX_BLK, L), lambda i: (i, 0)),
        pl.BlockSpec((_IDX_BLK, L), lambda i: (i, 0)),
    ],
    out_specs=pl.BlockSpec((L, _IDX_BLK), lambda i: (0, i)),
    out_shape=jax.ShapeDtypeStruct((L, B), jnp.int32),
)

_NC = 2
_NS = 16
_NW = _NC * _NS
_CHUNK = 4096


@functools.lru_cache(maxsize=None)
def _make_sc_gather():
    mesh = plsc.VectorSubcoreMesh(
        core_axis_name="c", subcore_axis_name="s",
        num_cores=_NC, num_subcores=_NS)

    @functools.partial(
        pl.kernel,
        mesh=mesh,
        out_type=jax.ShapeDtypeStruct((L, B // 128, D, 128), jnp.float32),
        scratch_types=[
            pltpu.VMEM((D, LUT_W), jnp.float32),
            pltpu.VMEM((2, B // 256, 128), jnp.int32),
            pltpu.VMEM((2, _CHUNK // 128, D, 128), jnp.float32),
            pltpu.SemaphoreType.DMA,
            pltpu.SemaphoreType.DMA,
        ],
        compiler_params=pltpu.CompilerParams(
            use_tc_tiling_on_sc=False, needs_layout_passes=False),
    )
    def _sc_gather(idx_hbm, lutc_hbm, out_hbm, lutc_v, idx_v2, out_v2,
                   isem, osem):
        wid = lax.axis_index("s") * _NC + lax.axis_index("c")
        u_start = 12 * wid + jnp.minimum(wid, 16)
        n_u = 12 + (wid < 16).astype(jnp.int32)
        nbt = B // 256
        pltpu.sync_copy(lutc_hbm, lutc_v)
        c_vecs = [jnp.zeros((16,), jnp.int32) + c for c in range(D)]

        def idx_copy(u, buf):
            l = u // 2
            h = u % 2
            return pltpu.make_async_copy(
                idx_hbm.at[l // D, pl.ds(h * nbt, nbt), l % D, :],
                idx_v2.at[buf], isem)

        idx_copy(u_start, 0).start()

        def u_body(ui, carry):
            u = u_start + ui
            l = u // 2
            h = u % 2
            idx_copy(u, ui % 2).wait()

            @pl.when(ui + 1 < n_u)
            def _():
                idx_copy(u + 1, (ui + 1) % 2).start()

            idxbuf = idx_v2.at[ui % 2]
            nt = _CHUNK // 128
            nch = B // (2 * _CHUNK)
            for ch in range(nch):
                g = ui * nch + ch
                obuf = out_v2.at[g % 2]

                @pl.when(g >= 2)
                def _():
                    pltpu.make_async_copy(
                        out_hbm.at[0, pl.ds(0, nt)],
                        out_v2.at[g % 2], osem).wait()

                kk0 = ch * (_CHUNK // 16)

                @plsc.parallel_loop(0, _CHUNK // 16, unroll=8)
                def _(k):
                    kk = kk0 + k
                    vec = idxbuf[kk // 8, pl.ds((kk % 8) * 16, 16)]
                    for c in range(D):
                        obuf[k // 8, c, pl.ds((k % 8) * 16, 16)] = (
                            plsc.load_gather(lutc_v, [c_vecs[c], vec]))
                pltpu.make_async_copy(
                    obuf,
                    out_hbm.at[l, pl.ds(h * nbt + ch * nt, nt)],
                    osem).start()
            return carry

        lax.fori_loop(0, n_u, u_body, 0)
        for _ in range(2):
            pltpu.make_async_copy(
                out_hbm.at[0, pl.ds(0, _CHUNK // 128)],
                out_v2.at[0], osem).wait()

    return _sc_gather


def kernel(input_1, input_2, table1, table2, W, b):
    i1 = input_1.astype(jnp.int32)
    i2 = input_2.astype(jnp.int32)
    lutc = _lutc_call(table1, table2, W, b.reshape(1, D))
    idxT = _idxT_call(i1, i2)
    idx4 = idxT.reshape(L // D, D, B // 128, 128).transpose(0, 2, 1, 3)
    out4 = _make_sc_gather()(idx4, lutc)
    return out4.transpose(1, 3, 0, 2).reshape(B, L, D)
